# baseline (speedup 1.0000x reference)
import jax
import jax.numpy as jnp
from jax import lax
from jax.experimental import pallas as pl
from jax.experimental.pallas import tpu as pltpu
from jax.experimental.pallas import tpu_sc as plsc

_FIELD_DIM = 38461
_NF = 26
_CONT = 13
_BATCH = 16384
_VOCAB = _FIELD_DIM * _NF
_NUM_CORES = 2
_NW = 32
_BPW = _BATCH // _NW
_L = 16
_BLK = 128
_NBLK = _BPW // _BLK
_BE = _BLK * _NF
_IPW = _BPW * _NF

_VMAIN = (_VOCAB // 1024) * 1024
_VTAIL = _VOCAB - _VMAIN
_TAILPAD = 1024
_CLAMP = _VMAIN - 1


def _idx_body(xt_h, idx_out_h, xt_v, idx_v, sem_in):
    c = lax.axis_index("c")
    s = lax.axis_index("s")
    wid = s * _NUM_CORES + c
    base = wid * _BPW

    pltpu.sync_copy(xt_h.at[:, pl.ds(base, _BPW)], xt_v)

    for j in range(_NBLK):
        jo = j * _BLK

        @pl.loop(0, _NF)
        def _idx_build(f, jo=jo, j=j):
            fo = f * _FIELD_DIM
            o = j * _BE + f * _BLK
            for k in range(_BLK // _L):
                xv = xt_v[f, pl.ds(jo + k * _L, _L)]
                idx_v[pl.ds(o + k * _L, _L)] = jnp.minimum(xv + fo, _CLAMP)

    pltpu.sync_copy(idx_v, idx_out_h.at[pl.ds(wid * _IPW, _IPW)])


def _gather_body(idx_h, xt_h, xct_h, table_h, tail_h, bias_h, w_h, out_h,
                 idx_v, xct_v, g_v, x25_v, tail_v, w_v, b_v, out_v,
                 sem_idx, sem_xc, sem0, sem1, sem2, sem3):
    c = lax.axis_index("c")
    s = lax.axis_index("s")
    wid = s * _NUM_CORES + c
    base = wid * _BPW
    sems = [sem0, sem1, sem2, sem3]

    idx_copy = pltpu.make_async_copy(
        idx_h.at[pl.ds(wid * _IPW, _IPW)], idx_v, sem_idx)
    idx_copy.start()
    xc_copy = pltpu.make_async_copy(
        xct_h.at[:, pl.ds(base, _BPW)], xct_v, sem_xc)
    xc_copy.start()
    pltpu.sync_copy(xt_h.at[_NF - 1, pl.ds(base, _BPW)], x25_v)
    pltpu.sync_copy(w_h, w_v.at[pl.ds(0, _CONT)])
    pltpu.sync_copy(bias_h, b_v.at[pl.ds(0, 1)])
    pltpu.sync_copy(tail_h, tail_v)
    idx_copy.wait()

    for j in range(_NBLK):
        pltpu.make_async_copy(
            table_h.at[idx_v.at[pl.ds(j * _BE, _BE)]],
            g_v.at[pl.ds(j * _BE, _BE)], sems[j],
        ).start()

    w_vec = w_v[...]
    w_s = [w_vec[i] for i in range(_CONT)]
    bias_s = b_v[...][0]
    xc_copy.wait()

    for j in range(_NBLK):
        jo = j * _BLK

        pltpu.make_async_copy(
            table_h.at[idx_v.at[pl.ds(j * _BE, _BE)]],
            g_v.at[pl.ds(j * _BE, _BE)], sems[j],
        ).wait()

        @pl.loop(0, _BLK // _L)
        def _acc_loop(k, jo=jo, j=j):
            o = jo + k * _L
            go = j * _BE + k * _L
            acc = jnp.full((_L,), bias_s, jnp.float32)
            for f in range(_NF - 1):
                acc = acc + g_v[pl.ds(go + f * _BLK, _L)]
            raw25 = x25_v[pl.ds(o, _L)] + (_NF - 1) * _FIELD_DIM
            toff = jnp.clip(raw25 - _CLAMP, 0, _TAILPAD - 1)
            tval = plsc.load_gather(tail_v, [toff])
            gval = g_v[pl.ds(go + (_NF - 1) * _BLK, _L)]
            acc = acc + jnp.where(raw25 > _CLAMP, tval, gval)
            for cc in range(_CONT):
                acc = acc + xct_v[cc, pl.ds(o, _L)] * w_s[cc]
            out_v[pl.ds(o, _L)] = acc

    pltpu.sync_copy(out_v, out_h.at[pl.ds(base, _BPW)])


def _make_kernels():
    mesh = plsc.VectorSubcoreMesh(core_axis_name="c", subcore_axis_name="s")
    idx_k = pl.kernel(
        _idx_body,
        out_type=jax.ShapeDtypeStruct((_BATCH * _NF,), jnp.int32),
        mesh=mesh,
        scratch_types=[
            pltpu.VMEM((_NF, _BPW), jnp.int32),
            pltpu.VMEM((_IPW,), jnp.int32),
            pltpu.SemaphoreType.DMA,
        ],
        compiler_params=pltpu.CompilerParams(needs_layout_passes=False),
    )
    gather_k = pl.kernel(
        _gather_body,
        out_type=jax.ShapeDtypeStruct((_BATCH,), jnp.float32),
        mesh=mesh,
        scratch_types=[
            pltpu.VMEM((_IPW,), jnp.int32),
            pltpu.VMEM((_CONT, _BPW), jnp.float32),
            pltpu.VMEM((_IPW,), jnp.float32),
            pltpu.VMEM((_BPW,), jnp.int32),
            pltpu.VMEM((_TAILPAD,), jnp.float32),
            pltpu.VMEM((_L,), jnp.float32),
            pltpu.VMEM((_L,), jnp.float32),
            pltpu.VMEM((_BPW,), jnp.float32),
            pltpu.SemaphoreType.DMA,
            pltpu.SemaphoreType.DMA,
            pltpu.SemaphoreType.DMA,
            pltpu.SemaphoreType.DMA,
            pltpu.SemaphoreType.DMA,
            pltpu.SemaphoreType.DMA,
        ],
        compiler_params=pltpu.CompilerParams(needs_layout_passes=False),
    )
    return idx_k, gather_k


_idx_sc, _gather_sc = _make_kernels()


@jax.jit
def kernel(x, x_cont, table, bias, w):
    xt = x.T
    tab_main = table[:_VMAIN, :].reshape(-1)
    tab_tail = jnp.pad(
        table[_CLAMP:, :], ((0, _TAILPAD - _VTAIL - 1), (0, 0))
    ).reshape(-1)
    idx = _idx_sc(xt)
    out = _gather_sc(idx, xt, x_cont.T, tab_main, tab_tail, bias, w)
    return out.reshape(-1, 1)


# speedup vs baseline: 1.1445x; 1.1445x over previous
import jax
import jax.numpy as jnp
from jax import lax
from jax.experimental import pallas as pl
from jax.experimental.pallas import tpu as pltpu
from jax.experimental.pallas import tpu_sc as plsc

_FIELD_DIM = 38461
_NF = 26
_CONT = 13
_BATCH = 16384
_VOCAB = _FIELD_DIM * _NF
_NUM_CORES = 2
_NW = 32
_BPW = _BATCH // _NW
_L = 16
_CHUNK = 128
_BLK = 128
_NBLK = _BPW // _BLK


_VMAIN = (_VOCAB // 1024) * 1024
_VTAIL = _VOCAB - _VMAIN
_TAILPAD = 1024


def _sc_body(xt_h, xct_h, table_h, tail_h, bias_h, w_h, out_h,
             xt_v, xct_v, idx_v, g_v, tail_v, w_v, b_v, out_v,
             sem_in, sem_xc, sem0, sem1, sem2, sem3):
    c = lax.axis_index("c")
    s = lax.axis_index("s")
    wid = s * _NUM_CORES + c
    base = wid * _BPW
    sems = [sem0, sem1, sem2, sem3]

    in_copy = pltpu.make_async_copy(
        xt_h.at[:, pl.ds(base, _BPW)], xt_v, sem_in)
    in_copy.start()
    xc_copy = pltpu.make_async_copy(
        xct_h.at[:, pl.ds(base, _BPW)], xct_v, sem_xc)
    xc_copy.start()
    pltpu.sync_copy(w_h, w_v.at[pl.ds(0, _CONT)])
    pltpu.sync_copy(bias_h, b_v.at[pl.ds(0, 1)])
    pltpu.sync_copy(tail_h, tail_v)
    in_copy.wait()

    for j in range(_NBLK):
        jo = j * _BLK

        @pl.loop(0, _NF)
        def _idx_fire(f, jo=jo, sem=sems[j]):
            fo = f * _FIELD_DIM
            o = f * _BPW + jo
            for k in range(_BLK // _L):
                xv = xt_v[f, pl.ds(jo + k * _L, _L)]
                idx_v[pl.ds(o + k * _L, _L)] = jnp.minimum(
                    xv + fo, _VMAIN - 1)
            pltpu.make_async_copy(
                table_h.at[idx_v.at[pl.ds(o, _CHUNK)]],
                g_v.at[pl.ds(o, _CHUNK)], sem,
            ).start()

    w_vec = w_v[...]
    w_s = [w_vec[i] for i in range(_CONT)]
    bias_s = b_v[...][0]
    xc_copy.wait()

    for j in range(_NBLK):
        jo = j * _BLK

        @pl.loop(0, _NF)
        def _drain(f, jo=jo, sem=sems[j]):
            o = f * _BPW + jo
            pltpu.make_async_copy(
                table_h.at[idx_v.at[pl.ds(o, _CHUNK)]],
                g_v.at[pl.ds(o, _CHUNK)], sem,
            ).wait()

        @pl.loop(0, _BLK // _L)
        def _acc_loop(k, jo=jo):
            o = jo + k * _L
            acc = jnp.full((_L,), bias_s, jnp.float32)
            for f in range(_NF - 1):
                acc = acc + g_v[pl.ds(o + f * _BPW, _L)]
            raw25 = xt_v[_NF - 1, pl.ds(o, _L)] + (_NF - 1) * _FIELD_DIM
            toff = jnp.clip(raw25 - _VMAIN, 0, _TAILPAD - 1)
            tval = plsc.load_gather(tail_v, [toff])
            gval = g_v[pl.ds(o + (_NF - 1) * _BPW, _L)]
            acc = acc + jnp.where(raw25 >= _VMAIN, tval, gval)
            for cc in range(_CONT):
                acc = acc + xct_v[cc, pl.ds(o, _L)] * w_s[cc]
            out_v[pl.ds(o, _L)] = acc

    pltpu.sync_copy(out_v, out_h.at[pl.ds(base, _BPW)])


def _make_kernel():
    mesh = plsc.VectorSubcoreMesh(core_axis_name="c", subcore_axis_name="s")
    return pl.kernel(
        _sc_body,
        out_type=jax.ShapeDtypeStruct((_BATCH,), jnp.float32),
        mesh=mesh,
        scratch_types=[
            pltpu.VMEM((_NF, _BPW), jnp.int32),
            pltpu.VMEM((_CONT, _BPW), jnp.float32),
            pltpu.VMEM((_NF * _BPW,), jnp.int32),
            pltpu.VMEM((_NF * _BPW,), jnp.float32),
            pltpu.VMEM((_TAILPAD,), jnp.float32),
            pltpu.VMEM((_L,), jnp.float32),
            pltpu.VMEM((_L,), jnp.float32),
            pltpu.VMEM((_BPW,), jnp.float32),
            pltpu.SemaphoreType.DMA,
            pltpu.SemaphoreType.DMA,
            pltpu.SemaphoreType.DMA,
            pltpu.SemaphoreType.DMA,
            pltpu.SemaphoreType.DMA,
            pltpu.SemaphoreType.DMA,
        ],
        compiler_params=pltpu.CompilerParams(needs_layout_passes=False),
    )


_fm_linear_sc = _make_kernel()


@jax.jit
def kernel(x, x_cont, table, bias, w):
    tab_main = table[:_VMAIN, :].reshape(-1)
    tab_tail = jnp.pad(
        table[_VMAIN:, :], ((0, _TAILPAD - _VTAIL), (0, 0))
    ).reshape(-1)
    out = _fm_linear_sc(x.T, x_cont.T, tab_main, tab_tail, bias, w)
    return out.reshape(-1, 1)
